# Initial kernel scaffold; baseline (speedup 1.0000x reference)
import jax
import jax.numpy as jnp
from jax.experimental import pallas as pl


def kernel(x, b, b2):
    raise NotImplementedError("write your pallas kernel here")


# speedup vs baseline: 2.3387x; 2.3387x over previous
import functools

import jax
import jax.numpy as jnp
from jax import lax
from jax.experimental import pallas as pl
from jax.experimental.pallas import tpu as pltpu
from jax.experimental.pallas import tpu_sc as plsc

_N = 16384
_A = 100
_EMB = 64
_E2 = 5
_E2P = 8
_TROWS = 2000
_D = 20.0

_NC, _NS, _L = 2, 16, 16
_NW = _NC * _NS
_CHUNK_ROWS = 2
_CHUNK = _CHUNK_ROWS * _A
_ROWS_PER_W = _N // _NW
_NCHUNK = _ROWS_PER_W // _CHUNK_ROWS
_GBATCH = 128
_NGB = 2
_GPAD = _NGB * _GBATCH


def _sc_body(x_hbm, t1_hbm, t2_hbm, h_hbm, h2_hbm,
             t1_sh, t2_sh, x_v, idx_v, t_v, g1_v, g2_v, oh_v, oh2_v, sem):
    c = lax.axis_index("c")
    s = lax.axis_index("s")

    del t1_sh, t2_sh

    wid = s * _NC + c
    base0 = wid * (_ROWS_PER_W * _A)

    iota = lax.iota(jnp.int32, _L)
    pair_col = iota - jnp.where(iota >= 8, 8, 0)
    pair_t = jnp.where(iota >= 8, 1, 0)

    def chunk_body(ci, carry):
        base = base0 + ci * _CHUNK
        pltpu.sync_copy(x_hbm.at[pl.ds(base, _CHUNK)], x_v)

        def idx_at(start):
            xv = x_v[pl.ds(start, _L)]
            xs = xv * _D
            xl = jnp.clip(xs.astype(jnp.int32), 0, 19)
            t = xs - xl.astype(jnp.float32)
            act = lax.rem(base + start + iota, _A)
            row = xl * _A + act
            idx_v[start // _GBATCH, pl.ds(start % _GBATCH, _L)] = row
            t_v[pl.ds(start, _L)] = t

        for start in [*range(0, _CHUNK - _L, _L), _CHUNK - _L]:
            idx_at(start)
        zeros16 = jnp.full((_L,), 0, jnp.int32)
        for start in [*range(_CHUNK, _GPAD - _L, _L), _GPAD - _L]:
            idx_v[start // _GBATCH, pl.ds(start % _GBATCH, _L)] = zeros16

        copies = []
        for j in range(_NGB):
            sl = pl.ds(j * _GBATCH, _GBATCH)
            copies.append(pltpu.make_async_copy(
                t1_hbm.at[idx_v.at[j]], g1_v.at[sl], sem))
            copies.append(pltpu.make_async_copy(
                t2_hbm.at[idx_v.at[j]], g2_v.at[sl], sem))
        for cp in copies:
            cp.start()
        for cp in copies:
            cp.wait()

        def h_body(e, carry2):
            tvec = plsc.load_gather(t_v, [jnp.full((_L,), 0, jnp.int32) + e])
            off_o = e * _EMB
            for j in range(_EMB // _L):
                lo = g1_v[e, pl.ds(j * _L, _L)]
                dv = g1_v[e, pl.ds(_EMB + j * _L, _L)]
                oh_v[pl.ds(off_o + j * _L, _L)] = lo + tvec * dv
            return carry2

        lax.fori_loop(0, _CHUNK, h_body, 0)

        def h2_body(e2, carry2):
            rows = e2 * 2 + pair_t
            lo = plsc.load_gather(g2_v, [rows, pair_col])
            dv = plsc.load_gather(g2_v, [rows, pair_col + _E2P])
            tvec = plsc.load_gather(t_v, [rows])
            oh2_v[pl.ds(e2 * 2 * _E2P, _L)] = lo + tvec * dv
            return carry2

        lax.fori_loop(0, _CHUNK // 2, h2_body, 0)

        pltpu.sync_copy(oh_v, h_hbm.at[pl.ds(base * _EMB, _CHUNK * _EMB)])
        pltpu.sync_copy(oh2_v, h2_hbm.at[pl.ds(base * _E2P, _CHUNK * _E2P)])
        return carry

    lax.fori_loop(0, _NCHUNK, chunk_body, 0)


_sc_call = pl.kernel(
    _sc_body,
    out_type=(
        jax.ShapeDtypeStruct((_N * _A * _EMB,), jnp.float32),
        jax.ShapeDtypeStruct((_N * _A * _E2P,), jnp.float32),
    ),
    mesh=plsc.VectorSubcoreMesh(core_axis_name="c", subcore_axis_name="s"),
    compiler_params=pltpu.CompilerParams(
        needs_layout_passes=False, use_tc_tiling_on_sc=False),
    scratch_types=[
        pltpu.VMEM_SHARED((_TROWS, 2 * _EMB), jnp.float32),
        pltpu.VMEM_SHARED((_TROWS, 2 * _E2P), jnp.float32),
        pltpu.VMEM((_CHUNK,), jnp.float32),
        pltpu.VMEM((_NGB, _GBATCH), jnp.int32),
        pltpu.VMEM((_CHUNK,), jnp.float32),
        pltpu.VMEM((_GPAD, 2 * _EMB), jnp.float32),
        pltpu.VMEM((_GPAD, 2 * _E2P), jnp.float32),
        pltpu.VMEM((_CHUNK * _EMB,), jnp.float32),
        pltpu.VMEM((_CHUNK * _E2P,), jnp.float32),
        pltpu.SemaphoreType.DMA,
    ],
)


def kernel(x, b, b2):
    zb = jnp.zeros((_A, _EMB), b.dtype)
    db = jnp.concatenate([b[_A:], zb], 0) - b
    t1 = jnp.concatenate([b, db], 1)[2000:2000 + _TROWS]

    b2p = jnp.pad(b2, ((0, 0), (0, _E2P - _E2)))
    zb2 = jnp.zeros((_A, _E2P), b2.dtype)
    db2 = jnp.concatenate([b2p[_A:], zb2], 0) - b2p
    t2 = jnp.concatenate([b2p, db2], 1)[2000:2000 + _TROWS]

    hf, h2f = _sc_call(x.reshape(-1), t1, t2)
    h = hf.reshape(_N, _A, _EMB)
    h2 = h2f.reshape(_N, _A, _E2P)[:, :, :_E2]
    return (h, h2)


# speedup vs baseline: 12.4213x; 5.3111x over previous
import jax
import jax.numpy as jnp
from jax import lax
from jax.experimental import pallas as pl
from jax.experimental.pallas import tpu as pltpu
from jax.experimental.pallas import tpu_sc as plsc

_N = 16384
_A = 100
_EMB = 64
_E2 = 5
_E2P = 8
_TROWS = 2112
_TMAX = 2099
_D = 20.0

_NC, _NS, _L = 2, 16, 16
_NW = _NC * _NS
_CHUNK_ROWS = 2
_CHUNK = _CHUNK_ROWS * _A
_ROWS_PER_W = _N // _NW
_NCHUNK = _ROWS_PER_W // _CHUNK_ROWS


def _sc_body(x_hbm, t1_hbm, t2_hbm, h_hbm, h2_hbm,
             t1_sh, t2_sh, x_v, tl_v, th_v,
             lo1_v, hi1_v, lo2_v, hi2_v, oh_v, oh2_v, sem):
    c = lax.axis_index("c")
    s = lax.axis_index("s")

    nslab = _TROWS // _NS
    @pl.when(s == 0)
    def _stage():
        for piece in range(_NS):
            slab = pl.ds(piece * nslab, nslab)
            pltpu.sync_copy(t1_hbm.at[slab], lo1_v.at[pl.ds(0, nslab)])
            pltpu.sync_copy(lo1_v.at[pl.ds(0, nslab)], t1_sh.at[slab])
            pltpu.sync_copy(t2_hbm.at[slab], lo2_v.at[pl.ds(0, nslab)])
            pltpu.sync_copy(lo2_v.at[pl.ds(0, nslab)], t2_sh.at[slab])

    plsc.subcore_barrier()

    wid = s * _NC + c
    base0 = wid * (_ROWS_PER_W * _A)

    iota = lax.iota(jnp.int32, _L)
    pair_col = iota - jnp.where(iota >= 8, 8, 0)
    pair_t = jnp.where(iota >= 8, 1, 0)

    def chunk_body(ci, carry):
        base = base0 + ci * _CHUNK
        pltpu.sync_copy(x_hbm.at[pl.ds(base, _CHUNK)], x_v)

        copies = []
        for start in [*range(0, _CHUNK - _L, _L), _CHUNK - _L]:
            xv = x_v[pl.ds(start, _L)]
            xs = xv * _D
            xl_i = xs.astype(jnp.int32)
            xh_i = (xs + 1.0).astype(jnp.int32)
            wl = (xh_i.astype(jnp.float32) / _D - xv) * _D
            wh = (xv - xl_i.astype(jnp.float32) / _D) * _D
            act = lax.rem(base + start + iota, _A)
            row_l = jnp.clip(xl_i * _A + act, 0, _TMAX)
            row_h = jnp.clip(xh_i * _A + act, 0, _TMAX)
            tl_v[pl.ds(start, _L)] = wl
            th_v[pl.ds(start, _L)] = wh
            sl = pl.ds(start, _L)
            copies.append(pltpu.make_async_copy(
                t1_sh.at[row_l], lo1_v.at[sl], sem))
            copies.append(pltpu.make_async_copy(
                t1_sh.at[row_h], hi1_v.at[sl], sem))
            copies.append(pltpu.make_async_copy(
                t2_sh.at[row_l], lo2_v.at[sl], sem))
            copies.append(pltpu.make_async_copy(
                t2_sh.at[row_h], hi2_v.at[sl], sem))
        for cp in copies:
            cp.start()
        for cp in copies:
            cp.wait()

        def h_body(e, carry2):
            evec = jnp.full((_L,), 0, jnp.int32) + e
            wl = plsc.load_gather(tl_v, [evec])
            wh = plsc.load_gather(th_v, [evec])
            off_o = e * _EMB
            for j in range(_EMB // _L):
                lo = lo1_v[e, pl.ds(j * _L, _L)]
                hi = hi1_v[e, pl.ds(j * _L, _L)]
                oh_v[pl.ds(off_o + j * _L, _L)] = lo * wl + hi * wh
            return carry2

        lax.fori_loop(0, _CHUNK, h_body, 0)

        def h2_body(e2, carry2):
            rows = e2 * 2 + pair_t
            lo = plsc.load_gather(lo2_v, [rows, pair_col])
            hi = plsc.load_gather(hi2_v, [rows, pair_col])
            wl = plsc.load_gather(tl_v, [rows])
            wh = plsc.load_gather(th_v, [rows])
            oh2_v[pl.ds(e2 * 2 * _E2P, _L)] = lo * wl + hi * wh
            return carry2

        lax.fori_loop(0, _CHUNK // 2, h2_body, 0)

        pltpu.sync_copy(oh_v, h_hbm.at[pl.ds(base * _EMB, _CHUNK * _EMB)])
        pltpu.sync_copy(oh2_v, h2_hbm.at[pl.ds(base * _E2P, _CHUNK * _E2P)])
        return carry

    lax.fori_loop(0, _NCHUNK, chunk_body, 0)


_sc_call = pl.kernel(
    _sc_body,
    out_type=(
        jax.ShapeDtypeStruct((_N * _A * _EMB,), jnp.float32),
        jax.ShapeDtypeStruct((_N * _A * _E2P,), jnp.float32),
    ),
    mesh=plsc.VectorSubcoreMesh(core_axis_name="c", subcore_axis_name="s"),
    compiler_params=pltpu.CompilerParams(
        needs_layout_passes=False, use_tc_tiling_on_sc=False),
    scratch_types=[
        pltpu.VMEM_SHARED((_TROWS, _EMB), jnp.float32),
        pltpu.VMEM_SHARED((_TROWS, _E2P), jnp.float32),
        pltpu.VMEM((_CHUNK,), jnp.float32),
        pltpu.VMEM((_CHUNK,), jnp.float32),
        pltpu.VMEM((_CHUNK,), jnp.float32),
        pltpu.VMEM((_CHUNK, _EMB), jnp.float32),
        pltpu.VMEM((_CHUNK, _EMB), jnp.float32),
        pltpu.VMEM((_CHUNK, _E2P), jnp.float32),
        pltpu.VMEM((_CHUNK, _E2P), jnp.float32),
        pltpu.VMEM((_CHUNK * _EMB,), jnp.float32),
        pltpu.VMEM((_CHUNK * _E2P,), jnp.float32),
        pltpu.SemaphoreType.DMA,
    ],
)


def kernel(x, b, b2):
    t1 = jnp.pad(b[2000:], ((0, _TROWS - 2100), (0, 0)))
    t2 = jnp.pad(b2[2000:], ((0, _TROWS - 2100), (0, _E2P - _E2)))
    hf, h2f = _sc_call(x.reshape(-1), t1, t2)
    h = hf.reshape(_N, _A, _EMB)
    h2 = h2f.reshape(_N, _A, _E2P)[:, :, :_E2]
    return (h, h2)
